# baseline (speedup 1.0000x reference)
import jax
import jax.numpy as jnp
from jax import lax
from jax.experimental import pallas as pl
from jax.experimental.pallas import tpu as pltpu
from jax.experimental.pallas import tpu_sc as plsc

CE = 128
NW = 32
BN = 2000


def _leaky(v):
    return jnp.where(v >= 0, v, 0.2 * v)


def _pre_body(x_ref, w_ref, br_ref, rad_ref, y_ref, ybf_ref):
    xp = jnp.dot(x_ref[...], w_ref[...], preferred_element_type=jnp.float32)
    half = br_ref.shape[-1]
    rad_ref[...] = _leaky(xp[:, :half] + br_ref[...])
    y_ref[...] = xp[:, half:]
    ybf_ref[...] = xp[:, half:].astype(jnp.bfloat16)


def _pre_call(x, w_cat, b_r, half):
    n, c = x.shape
    grid = (n // BN,)
    return pl.pallas_call(
        _pre_body,
        grid=grid,
        in_specs=[
            pl.BlockSpec((BN, c), lambda i: (i, 0)),
            pl.BlockSpec((c, 2 * half), lambda i: (0, 0)),
            pl.BlockSpec((1, half), lambda i: (0, 0)),
        ],
        out_specs=[
            pl.BlockSpec((BN, half), lambda i: (i, 0)),
            pl.BlockSpec((BN, half), lambda i: (i, 0)),
            pl.BlockSpec((BN, half), lambda i: (i, 0)),
        ],
        out_shape=[
            jax.ShapeDtypeStruct((n, half), jnp.float32),
            jax.ShapeDtypeStruct((n, half), jnp.float32),
            jax.ShapeDtypeStruct((n, half), jnp.bfloat16),
        ],
    )(x, w_cat, b_r.reshape(1, half))


def _sc_sum_call(y, src1d, dst1d, zsum, npad, cpw0, cpw1):
    half = y.shape[1]
    rpt = npad // 16
    cmax = max(cpw0, cpw1)

    mesh = plsc.VectorSubcoreMesh(core_axis_name="c", subcore_axis_name="s")

    def body(y_hbm, src_hbm, dst_hbm, zsum_hbm, out_sum,
             srcb, dstb, rows0, rows1, acc_s, gsem):
        c = lax.axis_index("c")
        s = lax.axis_index("s")
        pltpu.sync_copy(zsum_hbm.at[pl.ds(s * rpt, rpt)],
                        acc_s.at[pl.ds(s * rpt, rpt)])

        rows = (rows0, rows1)

        def run(base, cpw):
            pltpu.sync_copy(src_hbm.at[pl.ds(base, cpw)],
                            srcb.at[pl.ds(0, cpw)])
            pltpu.sync_copy(dst_hbm.at[pl.ds(base, cpw)],
                            dstb.at[pl.ds(0, cpw)])
            plsc.subcore_barrier()
            pltpu.async_copy(y_hbm.at[srcb.at[0]], rows0, gsem)
            pltpu.async_copy(y_hbm.at[srcb.at[1]], rows1, gsem)

            def step(i, carry):
                for b in range(2):
                    j = 2 * i + b
                    pltpu.make_async_copy(y_hbm.at[srcb.at[j]], rows[b],
                                          gsem).wait()
                    pltpu.sync_copy(rows[b], acc_s.at[dstb.at[j]], add=True)

                    @pl.when(j + 2 < cpw)
                    def _():
                        pltpu.async_copy(y_hbm.at[srcb.at[j + 2]], rows[b],
                                         gsem)

                return carry

            lax.fori_loop(0, cpw // 2, step, 0)

        @pl.when(c == 0)
        def _():
            run(s * cpw0, cpw0)

        @pl.when(c == 1)
        def _():
            run(16 * cpw0 + s * cpw1, cpw1)

        plsc.subcore_barrier()
        pltpu.sync_copy(acc_s.at[pl.ds(s * rpt, rpt)],
                        out_sum.at[pl.ds(c * npad + s * rpt, rpt)])

    fn = pl.kernel(
        body,
        out_type=jax.ShapeDtypeStruct((2 * npad, half), jnp.bfloat16),
        mesh=mesh,
        compiler_params=pltpu.CompilerParams(use_tc_tiling_on_sc=False),
        scratch_types=[
            pltpu.VMEM((cmax, CE), jnp.int32),
            pltpu.VMEM((cmax, CE), jnp.int32),
            pltpu.VMEM((CE, half), jnp.bfloat16),
            pltpu.VMEM((CE, half), jnp.bfloat16),
            pltpu.VMEM_SHARED((npad, half), jnp.bfloat16),
            pltpu.SemaphoreType.DMA,
        ],
    )
    return fn(y, src1d, dst1d, zsum)


def _sc_cnt_call(dst1d, zcnt1d, npad, epw):
    nps = npad // 16

    mesh = plsc.VectorSubcoreMesh(core_axis_name="c", subcore_axis_name="s")

    def body(dst_hbm, zcnt_hbm, out_cnt, dstl, cntl, res, stage):
        c = lax.axis_index("c")
        s = lax.axis_index("s")
        wid = s * 2 + c
        pltpu.sync_copy(zcnt_hbm, cntl)
        pltpu.sync_copy(dst_hbm.at[pl.ds(wid * epw, epw)], dstl)
        ones = jnp.ones((16,), jnp.float32)

        def step(i, carry):
            for b in range(8):
                dv = dstl[pl.ds((8 * i + b) * 16, 16)]
                plsc.addupdate_scatter(cntl, [dv], ones)
            return carry

        lax.fori_loop(0, epw // 16 // 8, step, 0)
        pltpu.sync_copy(cntl, stage.at[s])
        plsc.subcore_barrier()
        for t in range(16):
            pltpu.sync_copy(stage.at[t, pl.ds(s * nps, nps)],
                            cntl.at[pl.ds(t * nps, nps)])
        for k in range(nps // 16):
            acc = cntl[pl.ds(k * 16, 16)]
            for t in range(1, 16):
                acc = acc + cntl[pl.ds(t * nps + k * 16, 16)]
            res[pl.ds(k * 16, 16)] = acc
        pltpu.sync_copy(res, out_cnt.at[pl.ds(c * npad + s * nps, nps)])

    fn = pl.kernel(
        body,
        out_type=jax.ShapeDtypeStruct((2 * npad,), jnp.float32),
        mesh=mesh,
        compiler_params=pltpu.CompilerParams(needs_layout_passes=False),
        scratch_types=[
            pltpu.VMEM((epw,), jnp.int32),
            pltpu.VMEM((npad,), jnp.float32),
            pltpu.VMEM((npad // 16,), jnp.float32),
            pltpu.VMEM_SHARED((16, npad), jnp.float32),
        ],
    )
    return fn(dst1d, zcnt1d)


def _post_body(rad_ref, y_ref, s0_ref, s1_ref, cnt_ref,
               wf_ref, bc_ref, bf_ref, g_ref, b_ref, out_ref):
    half = y_ref.shape[-1]
    cnt = cnt_ref[...] + 1.0
    agg = (s0_ref[0].astype(jnp.float32) + s1_ref[0].astype(jnp.float32)
           + y_ref[...]) / cnt
    con = _leaky(agg + bc_ref[...])
    h = jnp.dot(rad_ref[...], wf_ref[:half, :],
                preferred_element_type=jnp.float32)
    h = h + jnp.dot(con, wf_ref[half:, :], preferred_element_type=jnp.float32)
    h = h + bf_ref[...]
    mean = jnp.mean(h, axis=-1, keepdims=True)
    zc = h - mean
    var = jnp.mean(zc * zc, axis=-1, keepdims=True)
    hn = zc * lax.rsqrt(var + 1e-5) * g_ref[...] + b_ref[...]
    out_ref[...] = _leaky(hn)


def _post_call(radial, y, sums, cnt_col, W_f, b_c, b_f, gamma, beta):
    n, half = y.shape
    out_ch = W_f.shape[1]
    grid = (n // BN,)
    return pl.pallas_call(
        _post_body,
        grid=grid,
        in_specs=[
            pl.BlockSpec((BN, half), lambda i: (i, 0)),
            pl.BlockSpec((BN, half), lambda i: (i, 0)),
            pl.BlockSpec((1, BN, half), lambda i: (0, i, 0)),
            pl.BlockSpec((1, BN, half), lambda i: (1, i, 0)),
            pl.BlockSpec((BN, 1), lambda i: (i, 0)),
            pl.BlockSpec((2 * half, out_ch), lambda i: (0, 0)),
            pl.BlockSpec((1, half), lambda i: (0, 0)),
            pl.BlockSpec((1, out_ch), lambda i: (0, 0)),
            pl.BlockSpec((1, out_ch), lambda i: (0, 0)),
            pl.BlockSpec((1, out_ch), lambda i: (0, 0)),
        ],
        out_specs=pl.BlockSpec((BN, out_ch), lambda i: (i, 0)),
        out_shape=jax.ShapeDtypeStruct((n, out_ch), jnp.float32),
    )(radial, y, sums, sums, cnt_col, W_f,
      b_c.reshape(1, half), b_f.reshape(1, out_ch),
      gamma.reshape(1, out_ch), beta.reshape(1, out_ch))


def kernel(x, edge_index, W_r, b_r, W_c, b_c, W_f, b_f, gamma, beta):
    n, in_ch = x.shape
    half = W_r.shape[1]
    e = edge_index.shape[1]

    epad = ((e + NW * CE - 1) // (NW * CE)) * (NW * CE)
    npad = ((n + 16 * CE - 1) // (16 * CE)) * (16 * CE)
    cpw = epad // NW // CE
    epw = epad // NW
    cpw0 = (2 * cpw * 3 // 10) // 8 * 8
    cpw1 = 2 * cpw - cpw0

    src = edge_index[0].astype(jnp.int32)
    dst = edge_index[1].astype(jnp.int32)
    src1d = jnp.concatenate([src, jnp.zeros((epad - e,), jnp.int32)])
    pad_dst = n + jnp.arange(epad - e, dtype=jnp.int32) % (npad - n)
    dst1d = jnp.concatenate([dst, pad_dst])

    w_cat = jnp.concatenate([W_r, W_c], axis=1)
    radial, y, ybf = _pre_call(x, w_cat, b_r, half)

    zsum = jnp.zeros((npad, half), jnp.bfloat16)
    zcnt1d = jnp.zeros((npad,), jnp.float32)
    src2d = src1d.reshape(epad // CE, CE)
    dst2d = dst1d.reshape(epad // CE, CE)
    sums_flat = _sc_sum_call(ybf, src2d, dst2d, zsum, npad, cpw0, cpw1)
    cnts_flat = _sc_cnt_call(dst1d, zcnt1d, npad, epw)

    sums = sums_flat.reshape(2, npad, half)
    cnt_col = (cnts_flat[:npad] + cnts_flat[npad:])[:n][:, None]

    return _post_call(radial, y, sums, cnt_col, W_f, b_c, b_f, gamma, beta)


# speedup vs baseline: 1.0618x; 1.0618x over previous
import jax
import jax.numpy as jnp
from jax import lax
from jax.experimental import pallas as pl
from jax.experimental.pallas import tpu as pltpu
from jax.experimental.pallas import tpu_sc as plsc

CE = 128
NW = 32
BN = 2000


def _leaky(v):
    return jnp.where(v >= 0, v, 0.2 * v)


def _pre_body(x_ref, w_ref, br_ref, rad_ref, y_ref, ybf_ref):
    xp = jnp.dot(x_ref[...], w_ref[...], preferred_element_type=jnp.float32)
    half = br_ref.shape[-1]
    rad_ref[...] = _leaky(xp[:, :half] + br_ref[...])
    y_ref[...] = xp[:, half:]
    ybf_ref[...] = xp[:, half:].astype(jnp.bfloat16)


def _pre_call(x, w_cat, b_r, half):
    n, c = x.shape
    grid = (n // BN,)
    return pl.pallas_call(
        _pre_body,
        grid=grid,
        in_specs=[
            pl.BlockSpec((BN, c), lambda i: (i, 0)),
            pl.BlockSpec((c, 2 * half), lambda i: (0, 0)),
            pl.BlockSpec((1, half), lambda i: (0, 0)),
        ],
        out_specs=[
            pl.BlockSpec((BN, half), lambda i: (i, 0)),
            pl.BlockSpec((BN, half), lambda i: (i, 0)),
            pl.BlockSpec((BN, half), lambda i: (i, 0)),
        ],
        out_shape=[
            jax.ShapeDtypeStruct((n, half), jnp.float32),
            jax.ShapeDtypeStruct((n, half), jnp.float32),
            jax.ShapeDtypeStruct((n, half), jnp.bfloat16),
        ],
    )(x, w_cat, b_r.reshape(1, half))


def _sc_sum_call(y, src1d, dst1d, zsum, npad, cpw0, cpw1):
    half = y.shape[1]
    rpt = npad // 16
    cmax = max(cpw0, cpw1)

    mesh = plsc.VectorSubcoreMesh(core_axis_name="c", subcore_axis_name="s")

    def body(y_hbm, src_hbm, dst_hbm, zsum_hbm, out_sum,
             srcb, dstb, rows0, rows1, acc_s, gsem):
        c = lax.axis_index("c")
        s = lax.axis_index("s")
        pltpu.sync_copy(zsum_hbm.at[pl.ds(s * rpt, rpt)],
                        acc_s.at[pl.ds(s * rpt, rpt)])

        rows = (rows0, rows1)

        def run(base, cpw):
            pltpu.sync_copy(src_hbm.at[pl.ds(base, cpw)],
                            srcb.at[pl.ds(0, cpw)])
            pltpu.sync_copy(dst_hbm.at[pl.ds(base, cpw)],
                            dstb.at[pl.ds(0, cpw)])
            plsc.subcore_barrier()
            pltpu.async_copy(y_hbm.at[srcb.at[0]], rows0, gsem)
            pltpu.async_copy(y_hbm.at[srcb.at[1]], rows1, gsem)

            def step(i, carry):
                for b in range(2):
                    j = 2 * i + b
                    pltpu.make_async_copy(y_hbm.at[srcb.at[j]], rows[b],
                                          gsem).wait()
                    pltpu.sync_copy(rows[b], acc_s.at[dstb.at[j]], add=True)

                    @pl.when(j + 2 < cpw)
                    def _():
                        pltpu.async_copy(y_hbm.at[srcb.at[j + 2]], rows[b],
                                         gsem)

                return carry

            lax.fori_loop(0, cpw // 2, step, 0)

        @pl.when(c == 0)
        def _():
            run(s * cpw0, cpw0)

        @pl.when(c == 1)
        def _():
            run(16 * cpw0 + s * cpw1, cpw1)

        plsc.subcore_barrier()
        pltpu.sync_copy(acc_s.at[pl.ds(s * rpt, rpt)],
                        out_sum.at[pl.ds(c * npad + s * rpt, rpt)])

    fn = pl.kernel(
        body,
        out_type=jax.ShapeDtypeStruct((2 * npad, half), jnp.bfloat16),
        mesh=mesh,
        compiler_params=pltpu.CompilerParams(use_tc_tiling_on_sc=False),
        scratch_types=[
            pltpu.VMEM((cmax, CE), jnp.int32),
            pltpu.VMEM((cmax, CE), jnp.int32),
            pltpu.VMEM((CE, half), jnp.bfloat16),
            pltpu.VMEM((CE, half), jnp.bfloat16),
            pltpu.VMEM_SHARED((npad, half), jnp.bfloat16),
            pltpu.SemaphoreType.DMA,
        ],
    )
    return fn(y, src1d, dst1d, zsum)


def _sc_cnt_call(dst1d, zcnt1d, npad, epw):
    nps = npad // 16

    mesh = plsc.VectorSubcoreMesh(core_axis_name="c", subcore_axis_name="s")

    def body(dst_hbm, zcnt_hbm, out_cnt, dstl, cntl, res, stage):
        c = lax.axis_index("c")
        s = lax.axis_index("s")
        wid = s * 2 + c
        pltpu.sync_copy(zcnt_hbm, cntl)
        pltpu.sync_copy(dst_hbm.at[pl.ds(wid * epw, epw)], dstl)
        ones = jnp.ones((16,), jnp.float32)

        def step(i, carry):
            for b in range(8):
                dv = dstl[pl.ds((8 * i + b) * 16, 16)]
                plsc.addupdate_scatter(cntl, [dv], ones)
            return carry

        lax.fori_loop(0, epw // 16 // 8, step, 0)
        pltpu.sync_copy(cntl, stage.at[s])
        plsc.subcore_barrier()
        for t in range(16):
            pltpu.sync_copy(stage.at[t, pl.ds(s * nps, nps)],
                            cntl.at[pl.ds(t * nps, nps)])
        for k in range(nps // 16):
            acc = cntl[pl.ds(k * 16, 16)]
            for t in range(1, 16):
                acc = acc + cntl[pl.ds(t * nps + k * 16, 16)]
            res[pl.ds(k * 16, 16)] = acc
        pltpu.sync_copy(res, out_cnt.at[pl.ds(c * npad + s * nps, nps)])

    fn = pl.kernel(
        body,
        out_type=jax.ShapeDtypeStruct((2 * npad,), jnp.float32),
        mesh=mesh,
        compiler_params=pltpu.CompilerParams(needs_layout_passes=False),
        scratch_types=[
            pltpu.VMEM((epw,), jnp.int32),
            pltpu.VMEM((npad,), jnp.float32),
            pltpu.VMEM((npad // 16,), jnp.float32),
            pltpu.VMEM_SHARED((16, npad), jnp.float32),
        ],
    )
    return fn(dst1d, zcnt1d)


def _post_body(rad_ref, y_ref, s0_ref, s1_ref, cnt_ref,
               wf_ref, bc_ref, bf_ref, g_ref, b_ref, out_ref):
    half = y_ref.shape[-1]
    cnt = cnt_ref[...] + 1.0
    agg = (s0_ref[0].astype(jnp.float32) + s1_ref[0].astype(jnp.float32)
           + y_ref[...]) / cnt
    con = _leaky(agg + bc_ref[...])
    h = jnp.dot(rad_ref[...], wf_ref[:half, :],
                preferred_element_type=jnp.float32)
    h = h + jnp.dot(con, wf_ref[half:, :], preferred_element_type=jnp.float32)
    h = h + bf_ref[...]
    mean = jnp.mean(h, axis=-1, keepdims=True)
    zc = h - mean
    var = jnp.mean(zc * zc, axis=-1, keepdims=True)
    hn = zc * lax.rsqrt(var + 1e-5) * g_ref[...] + b_ref[...]
    out_ref[...] = _leaky(hn)


def _post_call(radial, y, sums, cnt_col, W_f, b_c, b_f, gamma, beta):
    n, half = y.shape
    out_ch = W_f.shape[1]
    grid = (n // BN,)
    return pl.pallas_call(
        _post_body,
        grid=grid,
        in_specs=[
            pl.BlockSpec((BN, half), lambda i: (i, 0)),
            pl.BlockSpec((BN, half), lambda i: (i, 0)),
            pl.BlockSpec((1, BN, half), lambda i: (0, i, 0)),
            pl.BlockSpec((1, BN, half), lambda i: (1, i, 0)),
            pl.BlockSpec((BN, 1), lambda i: (i, 0)),
            pl.BlockSpec((2 * half, out_ch), lambda i: (0, 0)),
            pl.BlockSpec((1, half), lambda i: (0, 0)),
            pl.BlockSpec((1, out_ch), lambda i: (0, 0)),
            pl.BlockSpec((1, out_ch), lambda i: (0, 0)),
            pl.BlockSpec((1, out_ch), lambda i: (0, 0)),
        ],
        out_specs=pl.BlockSpec((BN, out_ch), lambda i: (i, 0)),
        out_shape=jax.ShapeDtypeStruct((n, out_ch), jnp.float32),
    )(radial, y, sums, sums, cnt_col, W_f,
      b_c.reshape(1, half), b_f.reshape(1, out_ch),
      gamma.reshape(1, out_ch), beta.reshape(1, out_ch))


def kernel(x, edge_index, W_r, b_r, W_c, b_c, W_f, b_f, gamma, beta):
    n, in_ch = x.shape
    half = W_r.shape[1]
    e = edge_index.shape[1]

    epad = ((e + NW * CE - 1) // (NW * CE)) * (NW * CE)
    npad = ((n + 16 * CE - 1) // (16 * CE)) * (16 * CE)
    cpw = epad // NW // CE
    epw = epad // NW
    cpw1 = (2 * cpw * 3 // 10) // 8 * 8
    cpw0 = 2 * cpw - cpw1

    src = edge_index[0].astype(jnp.int32)
    dst = edge_index[1].astype(jnp.int32)
    src1d = jnp.concatenate([src, jnp.zeros((epad - e,), jnp.int32)])
    pad_dst = n + jnp.arange(epad - e, dtype=jnp.int32) % (npad - n)
    dst1d = jnp.concatenate([dst, pad_dst])

    w_cat = jnp.concatenate([W_r, W_c], axis=1)
    radial, y, ybf = _pre_call(x, w_cat, b_r, half)

    zsum = jnp.zeros((npad, half), jnp.bfloat16)
    zcnt1d = jnp.zeros((npad,), jnp.float32)
    src2d = src1d.reshape(epad // CE, CE)
    dst2d = dst1d.reshape(epad // CE, CE)
    sums_flat = _sc_sum_call(ybf, src2d, dst2d, zsum, npad, cpw0, cpw1)
    cnts_flat = _sc_cnt_call(dst1d, zcnt1d, npad, epw)

    sums = sums_flat.reshape(2, npad, half)
    cnt_col = (cnts_flat[:npad] + cnts_flat[npad:])[:n][:, None]

    return _post_call(radial, y, sums, cnt_col, W_f, b_c, b_f, gamma, beta)


# speedup vs baseline: 1.4647x; 1.3795x over previous
import jax
import jax.numpy as jnp
from jax import lax
from jax.experimental import pallas as pl
from jax.experimental.pallas import tpu as pltpu
from jax.experimental.pallas import tpu_sc as plsc

CE = 128
NW = 32
BN = 2000


def _leaky(v):
    return jnp.where(v >= 0, v, 0.2 * v)


def _pre_body(x_ref, w_ref, br_ref, rad_ref, y_ref, ybf_ref):
    xp = jnp.dot(x_ref[...], w_ref[...], preferred_element_type=jnp.float32)
    half = br_ref.shape[-1]
    rad_ref[...] = _leaky(xp[:, :half] + br_ref[...])
    y_ref[...] = xp[:, half:]
    ybf_ref[...] = xp[:, half:].astype(jnp.bfloat16)


def _pre_call(x, w_cat, b_r, half):
    n, c = x.shape
    grid = (n // BN,)
    return pl.pallas_call(
        _pre_body,
        grid=grid,
        in_specs=[
            pl.BlockSpec((BN, c), lambda i: (i, 0)),
            pl.BlockSpec((c, 2 * half), lambda i: (0, 0)),
            pl.BlockSpec((1, half), lambda i: (0, 0)),
        ],
        out_specs=[
            pl.BlockSpec((BN, half), lambda i: (i, 0)),
            pl.BlockSpec((BN, half), lambda i: (i, 0)),
            pl.BlockSpec((BN, half), lambda i: (i, 0)),
        ],
        out_shape=[
            jax.ShapeDtypeStruct((n, half), jnp.float32),
            jax.ShapeDtypeStruct((n, half), jnp.float32),
            jax.ShapeDtypeStruct((n, half), jnp.bfloat16),
        ],
    )(x, w_cat, b_r.reshape(1, half))


def _sc_sum_call(y, src1d, dst1d, zsum, npad, cpw0, cpw1):
    half = y.shape[1]
    rpt = npad // 16
    cmax = max(cpw0, cpw1)

    mesh = plsc.VectorSubcoreMesh(core_axis_name="c", subcore_axis_name="s")

    def body(y_hbm, src_hbm, dst_hbm, zsum_hbm, out_sum,
             srcb, dstb, rows0, rows1, acc_s, y_spm, gsem):
        c = lax.axis_index("c")
        s = lax.axis_index("s")
        pltpu.sync_copy(zsum_hbm.at[pl.ds(s * rpt, rpt)],
                        acc_s.at[pl.ds(s * rpt, rpt)])
        pltpu.sync_copy(y_hbm.at[pl.ds(s * rpt, rpt)],
                        y_spm.at[pl.ds(s * rpt, rpt)])

        rows = (rows0, rows1)

        def run(base, cpw):
            pltpu.sync_copy(src_hbm.at[pl.ds(base, cpw)],
                            srcb.at[pl.ds(0, cpw)])
            pltpu.sync_copy(dst_hbm.at[pl.ds(base, cpw)],
                            dstb.at[pl.ds(0, cpw)])
            plsc.subcore_barrier()
            pltpu.async_copy(y_spm.at[srcb.at[0]], rows0, gsem)
            pltpu.async_copy(y_spm.at[srcb.at[1]], rows1, gsem)

            def step(i, carry):
                for b in range(2):
                    j = 2 * i + b
                    pltpu.make_async_copy(y_spm.at[srcb.at[j]], rows[b],
                                          gsem).wait()
                    pltpu.sync_copy(rows[b], acc_s.at[dstb.at[j]], add=True)

                    @pl.when(j + 2 < cpw)
                    def _():
                        pltpu.async_copy(y_spm.at[srcb.at[j + 2]], rows[b],
                                         gsem)

                return carry

            lax.fori_loop(0, cpw // 2, step, 0)

        @pl.when(c == 0)
        def _():
            run(s * cpw0, cpw0)

        @pl.when(c == 1)
        def _():
            run(16 * cpw0 + s * cpw1, cpw1)

        plsc.subcore_barrier()
        pltpu.sync_copy(acc_s.at[pl.ds(s * rpt, rpt)],
                        out_sum.at[pl.ds(c * npad + s * rpt, rpt)])

    fn = pl.kernel(
        body,
        out_type=jax.ShapeDtypeStruct((2 * npad, half), jnp.bfloat16),
        mesh=mesh,
        compiler_params=pltpu.CompilerParams(use_tc_tiling_on_sc=False),
        scratch_types=[
            pltpu.VMEM((cmax, CE), jnp.int32),
            pltpu.VMEM((cmax, CE), jnp.int32),
            pltpu.VMEM((CE, half), jnp.bfloat16),
            pltpu.VMEM((CE, half), jnp.bfloat16),
            pltpu.VMEM_SHARED((npad, half), jnp.bfloat16),
            pltpu.VMEM_SHARED((npad, half), jnp.bfloat16),
            pltpu.SemaphoreType.DMA,
        ],
    )
    return fn(y, src1d, dst1d, zsum)


def _sc_cnt_call(dst1d, zcnt1d, npad, epw):
    nps = npad // 16

    mesh = plsc.VectorSubcoreMesh(core_axis_name="c", subcore_axis_name="s")

    def body(dst_hbm, zcnt_hbm, out_cnt, dstl, cntl, res, stage):
        c = lax.axis_index("c")
        s = lax.axis_index("s")
        wid = s * 2 + c
        pltpu.sync_copy(zcnt_hbm, cntl)
        pltpu.sync_copy(dst_hbm.at[pl.ds(wid * epw, epw)], dstl)
        ones = jnp.ones((16,), jnp.float32)

        def step(i, carry):
            for b in range(8):
                dv = dstl[pl.ds((8 * i + b) * 16, 16)]
                plsc.addupdate_scatter(cntl, [dv], ones)
            return carry

        lax.fori_loop(0, epw // 16 // 8, step, 0)
        pltpu.sync_copy(cntl, stage.at[s])
        plsc.subcore_barrier()
        for t in range(16):
            pltpu.sync_copy(stage.at[t, pl.ds(s * nps, nps)],
                            cntl.at[pl.ds(t * nps, nps)])
        for k in range(nps // 16):
            acc = cntl[pl.ds(k * 16, 16)]
            for t in range(1, 16):
                acc = acc + cntl[pl.ds(t * nps + k * 16, 16)]
            res[pl.ds(k * 16, 16)] = acc
        pltpu.sync_copy(res, out_cnt.at[pl.ds(c * npad + s * nps, nps)])

    fn = pl.kernel(
        body,
        out_type=jax.ShapeDtypeStruct((2 * npad,), jnp.float32),
        mesh=mesh,
        compiler_params=pltpu.CompilerParams(needs_layout_passes=False),
        scratch_types=[
            pltpu.VMEM((epw,), jnp.int32),
            pltpu.VMEM((npad,), jnp.float32),
            pltpu.VMEM((npad // 16,), jnp.float32),
            pltpu.VMEM_SHARED((16, npad), jnp.float32),
        ],
    )
    return fn(dst1d, zcnt1d)


def _post_body(rad_ref, y_ref, s0_ref, s1_ref, cnt_ref,
               wf_ref, bc_ref, bf_ref, g_ref, b_ref, out_ref):
    half = y_ref.shape[-1]
    cnt = cnt_ref[...] + 1.0
    agg = (s0_ref[0].astype(jnp.float32) + s1_ref[0].astype(jnp.float32)
           + y_ref[...]) / cnt
    con = _leaky(agg + bc_ref[...])
    h = jnp.dot(rad_ref[...], wf_ref[:half, :],
                preferred_element_type=jnp.float32)
    h = h + jnp.dot(con, wf_ref[half:, :], preferred_element_type=jnp.float32)
    h = h + bf_ref[...]
    mean = jnp.mean(h, axis=-1, keepdims=True)
    zc = h - mean
    var = jnp.mean(zc * zc, axis=-1, keepdims=True)
    hn = zc * lax.rsqrt(var + 1e-5) * g_ref[...] + b_ref[...]
    out_ref[...] = _leaky(hn)


def _post_call(radial, y, sums, cnt_col, W_f, b_c, b_f, gamma, beta):
    n, half = y.shape
    out_ch = W_f.shape[1]
    grid = (n // BN,)
    return pl.pallas_call(
        _post_body,
        grid=grid,
        in_specs=[
            pl.BlockSpec((BN, half), lambda i: (i, 0)),
            pl.BlockSpec((BN, half), lambda i: (i, 0)),
            pl.BlockSpec((1, BN, half), lambda i: (0, i, 0)),
            pl.BlockSpec((1, BN, half), lambda i: (1, i, 0)),
            pl.BlockSpec((BN, 1), lambda i: (i, 0)),
            pl.BlockSpec((2 * half, out_ch), lambda i: (0, 0)),
            pl.BlockSpec((1, half), lambda i: (0, 0)),
            pl.BlockSpec((1, out_ch), lambda i: (0, 0)),
            pl.BlockSpec((1, out_ch), lambda i: (0, 0)),
            pl.BlockSpec((1, out_ch), lambda i: (0, 0)),
        ],
        out_specs=pl.BlockSpec((BN, out_ch), lambda i: (i, 0)),
        out_shape=jax.ShapeDtypeStruct((n, out_ch), jnp.float32),
    )(radial, y, sums, sums, cnt_col, W_f,
      b_c.reshape(1, half), b_f.reshape(1, out_ch),
      gamma.reshape(1, out_ch), beta.reshape(1, out_ch))


def kernel(x, edge_index, W_r, b_r, W_c, b_c, W_f, b_f, gamma, beta):
    n, in_ch = x.shape
    half = W_r.shape[1]
    e = edge_index.shape[1]

    epad = ((e + NW * CE - 1) // (NW * CE)) * (NW * CE)
    npad = ((n + 16 * CE - 1) // (16 * CE)) * (16 * CE)
    cpw = epad // NW // CE
    epw = epad // NW
    cpw1 = (2 * cpw * 3 // 10) // 8 * 8
    cpw0 = 2 * cpw - cpw1

    src = edge_index[0].astype(jnp.int32)
    dst = edge_index[1].astype(jnp.int32)
    src1d = jnp.concatenate([src, jnp.zeros((epad - e,), jnp.int32)])
    pad_dst = n + jnp.arange(epad - e, dtype=jnp.int32) % (npad - n)
    dst1d = jnp.concatenate([dst, pad_dst])

    w_cat = jnp.concatenate([W_r, W_c], axis=1)
    radial, y, ybf = _pre_call(x, w_cat, b_r, half)

    zsum = jnp.zeros((npad, half), jnp.bfloat16)
    zcnt1d = jnp.zeros((npad,), jnp.float32)
    src2d = src1d.reshape(epad // CE, CE)
    dst2d = dst1d.reshape(epad // CE, CE)
    ypad = jnp.concatenate(
        [ybf, jnp.zeros((npad - n, half), jnp.bfloat16)])
    sums_flat = _sc_sum_call(ypad, src2d, dst2d, zsum, npad, cpw0, cpw1)
    cnts_flat = _sc_cnt_call(dst1d, zcnt1d, npad, epw)

    sums = sums_flat.reshape(2, npad, half)
    cnt_col = (cnts_flat[:npad] + cnts_flat[npad:])[:n][:, None]

    return _post_call(radial, y, sums, cnt_col, W_f, b_c, b_f, gamma, beta)


# speedup vs baseline: 1.5942x; 1.0884x over previous
import jax
import jax.numpy as jnp
from jax import lax
from jax.experimental import pallas as pl
from jax.experimental.pallas import tpu as pltpu
from jax.experimental.pallas import tpu_sc as plsc

CE = 128
NW = 32
BN = 2000


def _leaky(v):
    return jnp.where(v >= 0, v, 0.2 * v)


def _pre_body(x_ref, w_ref, br_ref, rad_ref, y_ref, ybf_ref):
    xp = jnp.dot(x_ref[...], w_ref[...], preferred_element_type=jnp.float32)
    half = br_ref.shape[-1]
    rad_ref[...] = _leaky(xp[:, :half] + br_ref[...])
    y_ref[...] = xp[:, half:]
    ybf_ref[...] = xp[:, half:].astype(jnp.bfloat16)


def _pre_call(x, w_cat, b_r, half):
    n, c = x.shape
    grid = (n // BN,)
    return pl.pallas_call(
        _pre_body,
        grid=grid,
        in_specs=[
            pl.BlockSpec((BN, c), lambda i: (i, 0)),
            pl.BlockSpec((c, 2 * half), lambda i: (0, 0)),
            pl.BlockSpec((1, half), lambda i: (0, 0)),
        ],
        out_specs=[
            pl.BlockSpec((BN, half), lambda i: (i, 0)),
            pl.BlockSpec((BN, half), lambda i: (i, 0)),
            pl.BlockSpec((BN, half), lambda i: (i, 0)),
        ],
        out_shape=[
            jax.ShapeDtypeStruct((n, half), jnp.float32),
            jax.ShapeDtypeStruct((n, half), jnp.float32),
            jax.ShapeDtypeStruct((n, half), jnp.bfloat16),
        ],
    )(x, w_cat, b_r.reshape(1, half))


def _sc_sum_call(y, src1d, dst1d, zsum, npad, cpw0, cpw1):
    half = y.shape[1]
    rpt = npad // 16
    cmax = max(cpw0, cpw1)

    mesh = plsc.VectorSubcoreMesh(core_axis_name="c", subcore_axis_name="s")

    def body(y_hbm, src_hbm, dst_hbm, zsum_hbm, out_sum,
             srcb, dstb, rows0, rows1, acc_s, y_spm, gsem):
        c = lax.axis_index("c")
        s = lax.axis_index("s")
        pltpu.sync_copy(zsum_hbm.at[pl.ds(s * rpt, rpt)],
                        acc_s.at[pl.ds(s * rpt, rpt)])
        pltpu.sync_copy(y_hbm.at[pl.ds(s * rpt, rpt)],
                        y_spm.at[pl.ds(s * rpt, rpt)])

        rows = (rows0, rows1)

        def run(base, cpw):
            pltpu.sync_copy(src_hbm.at[pl.ds(base, cpw)],
                            srcb.at[pl.ds(0, cpw)])
            pltpu.sync_copy(dst_hbm.at[pl.ds(base, cpw)],
                            dstb.at[pl.ds(0, cpw)])
            plsc.subcore_barrier()
            pltpu.async_copy(y_spm.at[srcb.at[0]], rows0, gsem)
            pltpu.async_copy(y_spm.at[srcb.at[1]], rows1, gsem)

            def step(i, carry):
                for b in range(2):
                    j = 2 * i + b
                    pltpu.make_async_copy(y_spm.at[srcb.at[j]], rows[b],
                                          gsem).wait()
                    pltpu.sync_copy(rows[b], acc_s.at[dstb.at[j]], add=True)

                    @pl.when(j + 2 < cpw)
                    def _():
                        pltpu.async_copy(y_spm.at[srcb.at[j + 2]], rows[b],
                                         gsem)

                return carry

            lax.fori_loop(0, cpw // 2, step, 0)

        @pl.when(c == 0)
        def _():
            run(s * cpw0, cpw0)

        @pl.when(c == 1)
        def _():
            run(16 * cpw0 + s * cpw1, cpw1)

        plsc.subcore_barrier()
        pltpu.sync_copy(acc_s.at[pl.ds(s * rpt, rpt)],
                        out_sum.at[pl.ds(c * npad + s * rpt, rpt)])

    fn = pl.kernel(
        body,
        out_type=jax.ShapeDtypeStruct((2 * npad, half), jnp.bfloat16),
        mesh=mesh,
        compiler_params=pltpu.CompilerParams(use_tc_tiling_on_sc=False),
        scratch_types=[
            pltpu.VMEM((cmax, CE), jnp.int32),
            pltpu.VMEM((cmax, CE), jnp.int32),
            pltpu.VMEM((CE, half), jnp.bfloat16),
            pltpu.VMEM((CE, half), jnp.bfloat16),
            pltpu.VMEM_SHARED((npad, half), jnp.bfloat16),
            pltpu.VMEM_SHARED((npad, half), jnp.bfloat16),
            pltpu.SemaphoreType.DMA,
        ],
    )
    return fn(y, src1d, dst1d, zsum)


def _sc_cnt_call(dst1d, zcnt1d, npad, epw):
    nps = npad // 16

    mesh = plsc.VectorSubcoreMesh(core_axis_name="c", subcore_axis_name="s")

    def body(dst_hbm, zcnt_hbm, out_cnt, dstl, cntl, res, stage):
        c = lax.axis_index("c")
        s = lax.axis_index("s")
        wid = s * 2 + c
        pltpu.sync_copy(zcnt_hbm, cntl)
        pltpu.sync_copy(dst_hbm.at[pl.ds(wid * epw, epw)], dstl)
        ones = jnp.ones((16,), jnp.float32)

        def step(i, carry):
            for b in range(8):
                dv = dstl[pl.ds((8 * i + b) * 16, 16)]
                plsc.addupdate_scatter(cntl, [dv], ones)
            return carry

        lax.fori_loop(0, epw // 16 // 8, step, 0)
        pltpu.sync_copy(cntl, stage.at[s])
        plsc.subcore_barrier()
        for t in range(16):
            pltpu.sync_copy(stage.at[t, pl.ds(s * nps, nps)],
                            cntl.at[pl.ds(t * nps, nps)])
        for k in range(nps // 16):
            acc = cntl[pl.ds(k * 16, 16)]
            for t in range(1, 16):
                acc = acc + cntl[pl.ds(t * nps + k * 16, 16)]
            res[pl.ds(k * 16, 16)] = acc
        pltpu.sync_copy(res, out_cnt.at[pl.ds(c * npad + s * nps, nps)])

    fn = pl.kernel(
        body,
        out_type=jax.ShapeDtypeStruct((2 * npad,), jnp.float32),
        mesh=mesh,
        compiler_params=pltpu.CompilerParams(needs_layout_passes=False),
        scratch_types=[
            pltpu.VMEM((epw,), jnp.int32),
            pltpu.VMEM((npad,), jnp.float32),
            pltpu.VMEM((npad // 16,), jnp.float32),
            pltpu.VMEM_SHARED((16, npad), jnp.float32),
        ],
    )
    return fn(dst1d, zcnt1d)


def _post_body(rad_ref, y_ref, s0_ref, s1_ref, cnt_ref,
               wf_ref, bc_ref, bf_ref, g_ref, b_ref, out_ref):
    half = y_ref.shape[-1]
    cnt = cnt_ref[...] + 1.0
    agg = (s0_ref[0].astype(jnp.float32) + s1_ref[0].astype(jnp.float32)
           + y_ref[...]) / cnt
    con = _leaky(agg + bc_ref[...])
    h = jnp.dot(rad_ref[...], wf_ref[:half, :],
                preferred_element_type=jnp.float32)
    h = h + jnp.dot(con, wf_ref[half:, :], preferred_element_type=jnp.float32)
    h = h + bf_ref[...]
    mean = jnp.mean(h, axis=-1, keepdims=True)
    zc = h - mean
    var = jnp.mean(zc * zc, axis=-1, keepdims=True)
    hn = zc * lax.rsqrt(var + 1e-5) * g_ref[...] + b_ref[...]
    out_ref[...] = _leaky(hn)


def _post_call(radial, y, sums, cnt_col, W_f, b_c, b_f, gamma, beta):
    n, half = y.shape
    out_ch = W_f.shape[1]
    grid = (n // BN,)
    return pl.pallas_call(
        _post_body,
        grid=grid,
        in_specs=[
            pl.BlockSpec((BN, half), lambda i: (i, 0)),
            pl.BlockSpec((BN, half), lambda i: (i, 0)),
            pl.BlockSpec((1, BN, half), lambda i: (0, i, 0)),
            pl.BlockSpec((1, BN, half), lambda i: (1, i, 0)),
            pl.BlockSpec((BN, 1), lambda i: (i, 0)),
            pl.BlockSpec((2 * half, out_ch), lambda i: (0, 0)),
            pl.BlockSpec((1, half), lambda i: (0, 0)),
            pl.BlockSpec((1, out_ch), lambda i: (0, 0)),
            pl.BlockSpec((1, out_ch), lambda i: (0, 0)),
            pl.BlockSpec((1, out_ch), lambda i: (0, 0)),
        ],
        out_specs=pl.BlockSpec((BN, out_ch), lambda i: (i, 0)),
        out_shape=jax.ShapeDtypeStruct((n, out_ch), jnp.float32),
    )(radial, y, sums, sums, cnt_col, W_f,
      b_c.reshape(1, half), b_f.reshape(1, out_ch),
      gamma.reshape(1, out_ch), beta.reshape(1, out_ch))


def kernel(x, edge_index, W_r, b_r, W_c, b_c, W_f, b_f, gamma, beta):
    n, in_ch = x.shape
    half = W_r.shape[1]
    e = edge_index.shape[1]

    epad = ((e + NW * CE - 1) // (NW * CE)) * (NW * CE)
    npad = ((n + 16 * CE - 1) // (16 * CE)) * (16 * CE)
    cpw = epad // NW // CE
    epw = epad // NW
    cpw1 = cpw
    cpw0 = 2 * cpw - cpw1

    src = edge_index[0].astype(jnp.int32)
    dst = edge_index[1].astype(jnp.int32)
    src1d = jnp.concatenate([src, jnp.zeros((epad - e,), jnp.int32)])
    pad_dst = n + jnp.arange(epad - e, dtype=jnp.int32) % (npad - n)
    dst1d = jnp.concatenate([dst, pad_dst])

    w_cat = jnp.concatenate([W_r, W_c], axis=1)
    radial, y, ybf = _pre_call(x, w_cat, b_r, half)

    zsum = jnp.zeros((npad, half), jnp.bfloat16)
    zcnt1d = jnp.zeros((npad,), jnp.float32)
    src2d = src1d.reshape(epad // CE, CE)
    dst2d = dst1d.reshape(epad // CE, CE)
    ypad = jnp.concatenate(
        [ybf, jnp.zeros((npad - n, half), jnp.bfloat16)])
    sums_flat = _sc_sum_call(ypad, src2d, dst2d, zsum, npad, cpw0, cpw1)
    cnts_flat = _sc_cnt_call(dst1d, zcnt1d, npad, epw)

    sums = sums_flat.reshape(2, npad, half)
    cnt_col = (cnts_flat[:npad] + cnts_flat[npad:])[:n][:, None]

    return _post_call(radial, y, sums, cnt_col, W_f, b_c, b_f, gamma, beta)
